# baseline (speedup 1.0000x reference)
import functools

import jax
import jax.numpy as jnp
from jax import lax
from jax.experimental import pallas as pl
from jax.experimental.pallas import tpu as pltpu
from jax.experimental.pallas import tpu_sc as plsc

VOCAB = 1000000
N_EMBD = 64
BATCH = 4096
HIST = 200

BLK = 128
NB = BATCH * HIST // BLK
NW = 32
BPW = NB // NW
K = 5
NCHUNK = BPW // K

_mesh = plsc.VectorSubcoreMesh(core_axis_name="c", subcore_axis_name="s")


@functools.partial(
    pl.kernel,
    out_type=jax.ShapeDtypeStruct((NB, BLK, 2 * N_EMBD), jnp.float32),
    mesh=_mesh,
    scratch_types=[
        pltpu.VMEM((BPW, BLK), jnp.int32),
        pltpu.VMEM((K, BLK, N_EMBD), jnp.float32),
        pltpu.VMEM((K, BLK, N_EMBD), jnp.float32),
        pltpu.SemaphoreType.DMA,
        pltpu.SemaphoreType.DMA,
        pltpu.SemaphoreType.DMA,
        pltpu.SemaphoreType.DMA,
    ],
    compiler_params=pltpu.CompilerParams(use_tc_tiling_on_sc=False),
)
def _emb_lookup(x_hbm, w_hbm, out_hbm, idx_v, rows0, rows1, sg0, sg1, so0, so1):
    wid = lax.axis_index("s") * 2 + lax.axis_index("c")
    base0 = wid * BPW

    pltpu.sync_copy(x_hbm.at[wid], idx_v)

    rows = (rows0, rows1)
    sg = (sg0, sg1)
    so = (so0, so1)

    def fire_gathers(c, slot):
        for j in range(K):
            pltpu.async_copy(
                w_hbm.at[idx_v.at[c * K + j]], rows[slot].at[j], sg[slot]
            )

    def drain_gathers(slot):
        for j in range(K):
            pltpu.make_async_copy(
                w_hbm.at[idx_v.at[j]], rows[slot].at[j], sg[slot]
            ).wait()

    def out_slice(c):
        return out_hbm.at[pl.ds(base0 + c * K, K), :, pl.ds(0, N_EMBD)]

    def drain_out(slot):
        pltpu.make_async_copy(rows[slot], out_slice(0), so[slot]).wait()

    fire_gathers(0, 0)

    def body(i, carry):
        for b in range(2):
            c = 2 * i + b
            nxt = 1 - b
            @pl.when(c >= 1)
            def _():
                drain_out(nxt)

            @pl.when(c + 1 < NCHUNK)
            def _():
                fire_gathers(c + 1, nxt)

            drain_gathers(b)
            pltpu.async_copy(rows[b], out_slice(c), so[b])
        return carry

    lax.fori_loop(0, NCHUNK // 2, body, 0)
    drain_out(1)


def kernel(x, W):
    w_pad = jnp.concatenate([W, W], axis=1)
    w_flat = w_pad.reshape(2 * VOCAB, N_EMBD)
    x2 = (x.astype(jnp.int32) * 2).reshape(NW, BPW, BLK)
    out = _emb_lookup(x2, w_flat)
    return out.reshape(NB * BLK, 2 * N_EMBD)[:, :N_EMBD].reshape(
        BATCH, HIST, N_EMBD
    )


# speedup vs baseline: 1.1784x; 1.1784x over previous
import functools

import jax
import jax.numpy as jnp
from jax import lax
from jax.experimental import pallas as pl
from jax.experimental.pallas import tpu as pltpu
from jax.experimental.pallas import tpu_sc as plsc

VOCAB = 1000000
N_EMBD = 64
BATCH = 4096
HIST = 200

NTOK = BATCH * HIST
NW = 32
TPW = NTOK // NW
G = 640
NCHUNK = TPW // G
NCG = NTOK // G

_mesh = plsc.VectorSubcoreMesh(core_axis_name="c", subcore_axis_name="s")


@functools.partial(
    pl.kernel,
    out_type=jax.ShapeDtypeStruct((NCG, G, 2 * N_EMBD), jnp.float32),
    mesh=_mesh,
    scratch_types=[
        pltpu.VMEM((1, TPW), jnp.int32),
        pltpu.VMEM((G, N_EMBD), jnp.float32),
        pltpu.VMEM((G, N_EMBD), jnp.float32),
        pltpu.SemaphoreType.DMA,
        pltpu.SemaphoreType.DMA,
        pltpu.SemaphoreType.DMA,
        pltpu.SemaphoreType.DMA,
    ],
    compiler_params=pltpu.CompilerParams(use_tc_tiling_on_sc=False),
)
def _emb_lookup(x_hbm, w_hbm, out_hbm, idx_v, rows0, rows1, sg0, sg1, so0, so1):
    wid = lax.axis_index("s") * 2 + lax.axis_index("c")
    base0 = wid * NCHUNK

    pltpu.sync_copy(x_hbm.at[wid], idx_v)

    rows = (rows0, rows1)
    sg = (sg0, sg1)
    so = (so0, so1)

    def fire_gather(c, slot):
        pltpu.async_copy(
            w_hbm.at[idx_v.at[0, pl.ds(c * G, G)]], rows[slot], sg[slot]
        )

    def drain_gather(slot):
        pltpu.make_async_copy(
            w_hbm.at[idx_v.at[0, pl.ds(0, G)]], rows[slot], sg[slot]
        ).wait()

    def out_slice(c):
        return out_hbm.at[base0 + c, :, pl.ds(0, N_EMBD)]

    def drain_out(slot):
        pltpu.make_async_copy(rows[slot], out_slice(0), so[slot]).wait()

    fire_gather(0, 0)

    def body(i, carry):
        for b in range(2):
            c = 2 * i + b
            nxt = 1 - b
            @pl.when(c >= 1)
            def _():
                drain_out(nxt)

            @pl.when(c + 1 < NCHUNK)
            def _():
                fire_gather(c + 1, nxt)

            drain_gather(b)
            pltpu.async_copy(rows[b], out_slice(c), so[b])
        return carry

    lax.fori_loop(0, NCHUNK // 2, body, 0)
    drain_out(1)


def kernel(x, W):
    w_pad = jnp.pad(W, ((0, 0), (0, 128 - N_EMBD)))
    w_flat = w_pad.reshape(2 * VOCAB, N_EMBD)
    x2 = (x.astype(jnp.int32) * 2).reshape(NW, 1, TPW)
    out = _emb_lookup(x2, w_flat)
    return out.reshape(NTOK, 2 * N_EMBD)[:, :N_EMBD].reshape(
        BATCH, HIST, N_EMBD
    )
